# Initial kernel scaffold; baseline (speedup 1.0000x reference)
import jax
import jax.numpy as jnp
from jax.experimental import pallas as pl


def kernel(embeddings, output_weights, context, target, neg_samples):
    raise NotImplementedError("write your pallas kernel here")


# speedup vs baseline: 5.5834x; 5.5834x over previous
import functools

import jax
import jax.numpy as jnp
from jax import lax
from jax.experimental import pallas as pl
from jax.experimental.pallas import tpu as pltpu
from jax.experimental.pallas import tpu_sc as plsc

_NC = 2
_NS = 16
_NW = _NC * _NS
_L = 16

_IDXW = 80


@functools.cache
def _build_sc_scores(vocab, d, batch, ctx, neg):
    assert d % _L == 0
    bpw = batch // _NW
    unit = 16
    nunits = bpw // unit
    qn = d // _L
    assert (unit * ctx) % _IDXW == 0 and (unit * neg) % _IDXW == 0
    ctx_streams = unit * ctx // _IDXW
    neg_streams = unit * neg // _IDXW
    ctx_rows_w = bpw * ctx // _IDXW
    neg_rows_w = bpw * neg // _IDXW
    tgt_rows_w = bpw // 128

    mesh = plsc.VectorSubcoreMesh(core_axis_name="c", subcore_axis_name="s")

    @functools.partial(
        pl.kernel,
        out_type=(
            jax.ShapeDtypeStruct((batch,), jnp.float32),
            jax.ShapeDtypeStruct((batch * neg,), jnp.float32),
        ),
        mesh=mesh,
        compiler_params=pltpu.CompilerParams(needs_layout_passes=False,
                                             use_tc_tiling_on_sc=False),
        scratch_types=[
            pltpu.VMEM((ctx_rows_w, _IDXW), jnp.int32),
            pltpu.VMEM((neg_rows_w, _IDXW), jnp.int32),
            pltpu.VMEM((tgt_rows_w, 128), jnp.int32),
            pltpu.VMEM((bpw, d), jnp.float32),
            pltpu.VMEM((2, unit * ctx, d), jnp.float32),
            pltpu.VMEM((2, unit * neg, d), jnp.float32),
            pltpu.VMEM((bpw,), jnp.float32),
            pltpu.VMEM((nunits * neg * _L,), jnp.float32),
            pltpu.SemaphoreType.DMA,
            pltpu.SemaphoreType.DMA,
            pltpu.SemaphoreType.DMA,
        ],
    )
    def sc_scores(emb, ow, ctx_i, tgt_i, neg_i, pos_out, negt_out,
                  ctx_idx, neg_idx, tgt_idx, tgt_rows, ctx_buf, neg_buf,
                  pos_buf, negt_buf, sem_t, sem0, sem1):
        wid = lax.axis_index("s") * _NC + lax.axis_index("c")
        sems = (sem0, sem1)

        pltpu.sync_copy(ctx_i.at[pl.ds(wid * ctx_rows_w, ctx_rows_w)], ctx_idx)
        pltpu.sync_copy(neg_i.at[pl.ds(wid * neg_rows_w, neg_rows_w)], neg_idx)
        pltpu.sync_copy(tgt_i.at[pl.ds(wid * tgt_rows_w, tgt_rows_w)], tgt_idx)

        for j in range(tgt_rows_w):
            pltpu.async_copy(ow.at[tgt_idx.at[j]],
                             tgt_rows.at[pl.ds(j * 128, 128)], sem_t)
        for j in range(tgt_rows_w):
            pltpu.make_async_copy(ow.at[tgt_idx.at[j]],
                                  tgt_rows.at[pl.ds(j * 128, 128)],
                                  sem_t).wait()

        def fire(u, b):
            for j in range(ctx_streams):
                pltpu.async_copy(emb.at[ctx_idx.at[u * ctx_streams + j]],
                                 ctx_buf.at[b, pl.ds(j * _IDXW, _IDXW)],
                                 sems[b])
            for j in range(neg_streams):
                pltpu.async_copy(ow.at[neg_idx.at[u * neg_streams + j]],
                                 neg_buf.at[b, pl.ds(j * _IDXW, _IDXW)],
                                 sems[b])

        def drain(u, b):
            for j in range(ctx_streams):
                pltpu.make_async_copy(
                    emb.at[ctx_idx.at[u * ctx_streams + j]],
                    ctx_buf.at[b, pl.ds(j * _IDXW, _IDXW)], sems[b]).wait()
            for j in range(neg_streams):
                pltpu.make_async_copy(
                    ow.at[neg_idx.at[u * neg_streams + j]],
                    neg_buf.at[b, pl.ds(j * _IDXW, _IDXW)], sems[b]).wait()

        iota = lax.iota(jnp.int32, _L)

        def hsum(v):
            return plsc.cumsum(v)[_L - 1]

        def compute(u, b):
            def row_body(r, carry):
                pos_vec, nvecs = carry
                crow = r * ctx
                a = []
                for q in range(qn):
                    acc = ctx_buf[b, crow, pl.ds(q * _L, _L)]
                    for k in range(1, ctx):
                        acc = acc + ctx_buf[b, crow + k, pl.ds(q * _L, _L)]
                    a.append(acc * (1.0 / ctx))
                rg = u * unit + r
                e = a[0] * tgt_rows[rg, pl.ds(0, _L)]
                for q in range(1, qn):
                    e = e + a[q] * tgt_rows[rg, pl.ds(q * _L, _L)]
                pos_vec = jnp.where(iota == r, hsum(e), pos_vec)
                nrow = r * neg
                new_nvecs = []
                for n in range(neg):
                    e = a[0] * neg_buf[b, nrow + n, pl.ds(0, _L)]
                    for q in range(1, qn):
                        e = e + a[q] * neg_buf[b, nrow + n, pl.ds(q * _L, _L)]
                    new_nvecs.append(jnp.where(iota == r, hsum(e), nvecs[n]))
                return (pos_vec, tuple(new_nvecs))

            zero = jnp.zeros((_L,), jnp.float32)
            pos_vec, nvecs = lax.fori_loop(0, unit, row_body,
                                           (zero, (zero,) * neg))
            plsc.store_scatter(pos_buf, [u * unit + iota], pos_vec)
            for n in range(neg):
                plsc.store_scatter(negt_buf, [(u * neg + n) * _L + iota],
                                   nvecs[n])

        fire(0, 0)

        def pair_body(up, carry):
            for b in range(2):
                u = up * 2 + b

                @pl.when(u + 1 < nunits)
                def _fire_next():
                    fire(u + 1, 1 - b)

                drain(u, b)
                compute(u, b)
            return carry

        lax.fori_loop(0, nunits // 2, pair_body, 0)

        pltpu.sync_copy(pos_buf, pos_out.at[pl.ds(wid * bpw, bpw)])
        pltpu.sync_copy(negt_buf,
                        negt_out.at[pl.ds(wid * nunits * neg * _L,
                                          nunits * neg * _L)])

    return sc_scores


@functools.cache
def _build_tc_loss(batch, neg):
    def body(pos_ref, neg_ref, out_ref):
        p = pos_ref[...]
        s = neg_ref[...]
        sp_p = jnp.maximum(-p, 0.0) + jnp.log(1.0 + jnp.exp(-jnp.abs(p)))
        sp_n = jnp.maximum(s, 0.0) + jnp.log(1.0 + jnp.exp(-jnp.abs(s)))
        val = (jnp.sum(sp_p) * (1.0 / batch)
               + jnp.sum(sp_n) * (1.0 / (batch * neg)))
        out_ref[...] = val.reshape(1, 1)

    return pl.pallas_call(
        body,
        out_shape=jax.ShapeDtypeStruct((1, 1), jnp.float32),
    )


@jax.jit
def kernel(embeddings, output_weights, context, target, neg_samples):
    vocab, d = embeddings.shape
    batch, ctx = context.shape
    neg = neg_samples.shape[1]
    sc = _build_sc_scores(vocab, d, batch, ctx, neg)
    tc = _build_tc_loss(batch, neg)
    pos, negt = sc(embeddings, output_weights,
                   context.reshape(-1, _IDXW),
                   target.reshape(-1, 128),
                   neg_samples.reshape(-1, _IDXW))
    out = tc(pos.reshape(-1, 128), negt.reshape(-1, 128))
    return out[0, 0]


# speedup vs baseline: 5.6304x; 1.0084x over previous
import functools

import jax
import jax.numpy as jnp
from jax import lax
from jax.experimental import pallas as pl
from jax.experimental.pallas import tpu as pltpu
from jax.experimental.pallas import tpu_sc as plsc

_NC = 2
_NS = 16
_NW = _NC * _NS
_L = 16


@functools.cache
def _build_sc_scores(vocab, d, batch, ctx, neg):
    assert d % _L == 0
    bpw = batch // _NW
    unit = 16
    nunits = bpw // unit
    qn = d // _L

    mesh = plsc.VectorSubcoreMesh(core_axis_name="c", subcore_axis_name="s")

    @functools.partial(
        pl.kernel,
        out_type=(
            jax.ShapeDtypeStruct((batch,), jnp.float32),
            jax.ShapeDtypeStruct((batch * neg,), jnp.float32),
        ),
        mesh=mesh,
        compiler_params=pltpu.CompilerParams(needs_layout_passes=False,
                                             use_tc_tiling_on_sc=False),
        scratch_types=[
            pltpu.VMEM((ctx, bpw), jnp.int32),
            pltpu.VMEM((neg, bpw), jnp.int32),
            pltpu.VMEM((bpw,), jnp.int32),
            pltpu.VMEM((2, ctx, unit, d), jnp.float32),
            pltpu.VMEM((2, neg, unit, d), jnp.float32),
            pltpu.VMEM((2, unit, d), jnp.float32),
            pltpu.VMEM((bpw,), jnp.float32),
            pltpu.VMEM((nunits * neg * _L,), jnp.float32),
            pltpu.SemaphoreType.DMA,
            pltpu.SemaphoreType.DMA,
            pltpu.SemaphoreType.DMA,
        ],
    )
    def sc_scores(emb, ow, ctx_i, tgt_i, neg_i, pos_out, negt_out,
                  ctx_idx, neg_idx, tgt_idx, ctx_buf, neg_buf, tgt_buf,
                  pos_buf, negt_buf, sem_s, sem0, sem1):
        wid = lax.axis_index("s") * _NC + lax.axis_index("c")
        sems = (sem0, sem1)
        base = wid * bpw

        for k in range(ctx):
            pltpu.async_copy(ctx_i.at[k, pl.ds(base, bpw)], ctx_idx.at[k],
                             sem_s)
        for n in range(neg):
            pltpu.async_copy(neg_i.at[n, pl.ds(base, bpw)], neg_idx.at[n],
                             sem_s)
        pltpu.async_copy(tgt_i.at[pl.ds(base, bpw)], tgt_idx, sem_s)
        for k in range(ctx):
            pltpu.make_async_copy(ctx_i.at[k, pl.ds(base, bpw)],
                                  ctx_idx.at[k], sem_s).wait()
        for n in range(neg):
            pltpu.make_async_copy(neg_i.at[n, pl.ds(base, bpw)],
                                  neg_idx.at[n], sem_s).wait()
        pltpu.make_async_copy(tgt_i.at[pl.ds(base, bpw)], tgt_idx,
                              sem_s).wait()

        def fire(u, b):
            for k in range(ctx):
                pltpu.async_copy(emb.at[ctx_idx.at[k, pl.ds(u * unit, unit)]],
                                 ctx_buf.at[b, k], sems[b])
            for n in range(neg):
                pltpu.async_copy(ow.at[neg_idx.at[n, pl.ds(u * unit, unit)]],
                                 neg_buf.at[b, n], sems[b])
            pltpu.async_copy(ow.at[tgt_idx.at[pl.ds(u * unit, unit)]],
                             tgt_buf.at[b], sems[b])

        def drain(u, b):
            for k in range(ctx):
                pltpu.make_async_copy(
                    emb.at[ctx_idx.at[k, pl.ds(u * unit, unit)]],
                    ctx_buf.at[b, k], sems[b]).wait()
            for n in range(neg):
                pltpu.make_async_copy(
                    ow.at[neg_idx.at[n, pl.ds(u * unit, unit)]],
                    neg_buf.at[b, n], sems[b]).wait()
            pltpu.make_async_copy(ow.at[tgt_idx.at[pl.ds(u * unit, unit)]],
                                  tgt_buf.at[b], sems[b]).wait()

        iota = lax.iota(jnp.int32, _L)

        def hsum(v):
            return plsc.cumsum(v)[_L - 1]

        def compute(u, b):
            def row_body(r, carry):
                pos_vec, nvecs = carry
                a = []
                for q in range(qn):
                    acc = ctx_buf[b, 0, r, pl.ds(q * _L, _L)]
                    for k in range(1, ctx):
                        acc = acc + ctx_buf[b, k, r, pl.ds(q * _L, _L)]
                    a.append(acc * (1.0 / ctx))
                e = a[0] * tgt_buf[b, r, pl.ds(0, _L)]
                for q in range(1, qn):
                    e = e + a[q] * tgt_buf[b, r, pl.ds(q * _L, _L)]
                pos_vec = jnp.where(iota == r, hsum(e), pos_vec)
                new_nvecs = []
                for n in range(neg):
                    e = a[0] * neg_buf[b, n, r, pl.ds(0, _L)]
                    for q in range(1, qn):
                        e = e + a[q] * neg_buf[b, n, r, pl.ds(q * _L, _L)]
                    new_nvecs.append(jnp.where(iota == r, hsum(e), nvecs[n]))
                return (pos_vec, tuple(new_nvecs))

            zero = jnp.zeros((_L,), jnp.float32)
            pos_vec, nvecs = lax.fori_loop(0, unit, row_body,
                                           (zero, (zero,) * neg))
            plsc.store_scatter(pos_buf, [u * unit + iota], pos_vec)
            for n in range(neg):
                plsc.store_scatter(negt_buf, [(u * neg + n) * _L + iota],
                                   nvecs[n])

        fire(0, 0)

        def pair_body(up, carry):
            for b in range(2):
                u = up * 2 + b

                @pl.when(u + 1 < nunits)
                def _fire_next():
                    fire(u + 1, 1 - b)

                drain(u, b)
                compute(u, b)
            return carry

        lax.fori_loop(0, nunits // 2, pair_body, 0)

        pltpu.sync_copy(pos_buf, pos_out.at[pl.ds(base, bpw)])
        pltpu.sync_copy(negt_buf,
                        negt_out.at[pl.ds(wid * nunits * neg * _L,
                                          nunits * neg * _L)])

    return sc_scores


@functools.cache
def _build_tc_loss(batch, neg):
    def body(pos_ref, neg_ref, out_ref):
        p = pos_ref[...]
        s = neg_ref[...]
        sp_p = jnp.maximum(-p, 0.0) + jnp.log(1.0 + jnp.exp(-jnp.abs(p)))
        sp_n = jnp.maximum(s, 0.0) + jnp.log(1.0 + jnp.exp(-jnp.abs(s)))
        val = (jnp.sum(sp_p) * (1.0 / batch)
               + jnp.sum(sp_n) * (1.0 / (batch * neg)))
        out_ref[...] = val.reshape(1, 1)

    return pl.pallas_call(
        body,
        out_shape=jax.ShapeDtypeStruct((1, 1), jnp.float32),
    )


@jax.jit
def kernel(embeddings, output_weights, context, target, neg_samples):
    vocab, d = embeddings.shape
    batch, ctx = context.shape
    neg = neg_samples.shape[1]
    sc = _build_sc_scores(vocab, d, batch, ctx, neg)
    tc = _build_tc_loss(batch, neg)
    pos, negt = sc(embeddings, output_weights,
                   context.T, target, neg_samples.T)
    out = tc(pos.reshape(-1, 128), negt.reshape(-1, 128))
    return out[0, 0]


# speedup vs baseline: 5.6341x; 1.0007x over previous
import functools

import jax
import jax.numpy as jnp
from jax import lax
from jax.experimental import pallas as pl
from jax.experimental.pallas import tpu as pltpu
from jax.experimental.pallas import tpu_sc as plsc

_NC = 2
_NS = 16
_NW = _NC * _NS
_L = 16


@functools.cache
def _build_sc_scores(vocab, d, batch, ctx, neg):
    assert d % _L == 0
    bpw = batch // _NW
    unit = 16
    nunits = bpw // unit
    qn = d // _L

    mesh = plsc.VectorSubcoreMesh(core_axis_name="c", subcore_axis_name="s")

    @functools.partial(
        pl.kernel,
        out_type=(
            jax.ShapeDtypeStruct((batch,), jnp.float32),
            jax.ShapeDtypeStruct((batch * neg,), jnp.float32),
        ),
        mesh=mesh,
        compiler_params=pltpu.CompilerParams(needs_layout_passes=False,
                                             use_tc_tiling_on_sc=False),
        scratch_types=[
            pltpu.VMEM((ctx, bpw), jnp.int32),
            pltpu.VMEM((neg, bpw), jnp.int32),
            pltpu.VMEM((bpw,), jnp.int32),
            pltpu.VMEM((2, ctx, unit, d), jnp.float32),
            pltpu.VMEM((2, neg, unit, d), jnp.float32),
            pltpu.VMEM((2, unit, d), jnp.float32),
            pltpu.VMEM((bpw,), jnp.float32),
            pltpu.VMEM((nunits * neg * _L,), jnp.float32),
            pltpu.SemaphoreType.DMA,
            pltpu.SemaphoreType.DMA,
            pltpu.SemaphoreType.DMA,
        ],
    )
    def sc_scores(*refs):
        emb, ow = refs[0], refs[1]
        ctx_cols = refs[2:2 + ctx]
        tgt_i = refs[2 + ctx]
        neg_cols = refs[3 + ctx:3 + ctx + neg]
        (pos_out, negt_out, ctx_idx, neg_idx, tgt_idx, ctx_buf, neg_buf,
         tgt_buf, pos_buf, negt_buf, sem_s, sem0, sem1) = refs[3 + ctx + neg:]
        wid = lax.axis_index("s") * _NC + lax.axis_index("c")
        sems = (sem0, sem1)
        base = wid * bpw

        for k in range(ctx):
            pltpu.async_copy(ctx_cols[k].at[pl.ds(base, bpw)], ctx_idx.at[k],
                             sem_s)
        for n in range(neg):
            pltpu.async_copy(neg_cols[n].at[pl.ds(base, bpw)], neg_idx.at[n],
                             sem_s)
        pltpu.async_copy(tgt_i.at[pl.ds(base, bpw)], tgt_idx, sem_s)
        for k in range(ctx):
            pltpu.make_async_copy(ctx_cols[k].at[pl.ds(base, bpw)],
                                  ctx_idx.at[k], sem_s).wait()
        for n in range(neg):
            pltpu.make_async_copy(neg_cols[n].at[pl.ds(base, bpw)],
                                  neg_idx.at[n], sem_s).wait()
        pltpu.make_async_copy(tgt_i.at[pl.ds(base, bpw)], tgt_idx,
                              sem_s).wait()

        def fire(u, b):
            for k in range(ctx):
                pltpu.async_copy(emb.at[ctx_idx.at[k, pl.ds(u * unit, unit)]],
                                 ctx_buf.at[b, k], sems[b])
            for n in range(neg):
                pltpu.async_copy(ow.at[neg_idx.at[n, pl.ds(u * unit, unit)]],
                                 neg_buf.at[b, n], sems[b])
            pltpu.async_copy(ow.at[tgt_idx.at[pl.ds(u * unit, unit)]],
                             tgt_buf.at[b], sems[b])

        def drain(u, b):
            for k in range(ctx):
                pltpu.make_async_copy(
                    emb.at[ctx_idx.at[k, pl.ds(u * unit, unit)]],
                    ctx_buf.at[b, k], sems[b]).wait()
            for n in range(neg):
                pltpu.make_async_copy(
                    ow.at[neg_idx.at[n, pl.ds(u * unit, unit)]],
                    neg_buf.at[b, n], sems[b]).wait()
            pltpu.make_async_copy(ow.at[tgt_idx.at[pl.ds(u * unit, unit)]],
                                  tgt_buf.at[b], sems[b]).wait()

        iota = lax.iota(jnp.int32, _L)

        def hsum(v):
            return plsc.cumsum(v)[_L - 1]

        def compute(u, b):
            def row_body(r, carry):
                pos_vec, nvecs = carry
                a = []
                for q in range(qn):
                    acc = ctx_buf[b, 0, r, pl.ds(q * _L, _L)]
                    for k in range(1, ctx):
                        acc = acc + ctx_buf[b, k, r, pl.ds(q * _L, _L)]
                    a.append(acc * (1.0 / ctx))
                e = a[0] * tgt_buf[b, r, pl.ds(0, _L)]
                for q in range(1, qn):
                    e = e + a[q] * tgt_buf[b, r, pl.ds(q * _L, _L)]
                pos_vec = jnp.where(iota == r, hsum(e), pos_vec)
                new_nvecs = []
                for n in range(neg):
                    e = a[0] * neg_buf[b, n, r, pl.ds(0, _L)]
                    for q in range(1, qn):
                        e = e + a[q] * neg_buf[b, n, r, pl.ds(q * _L, _L)]
                    new_nvecs.append(jnp.where(iota == r, hsum(e), nvecs[n]))
                return (pos_vec, tuple(new_nvecs))

            zero = jnp.zeros((_L,), jnp.float32)
            pos_vec, nvecs = lax.fori_loop(0, unit, row_body,
                                           (zero, (zero,) * neg))
            plsc.store_scatter(pos_buf, [u * unit + iota], pos_vec)
            for n in range(neg):
                plsc.store_scatter(negt_buf, [(u * neg + n) * _L + iota],
                                   nvecs[n])

        fire(0, 0)

        def pair_body(up, carry):
            for b in range(2):
                u = up * 2 + b

                @pl.when(u + 1 < nunits)
                def _fire_next():
                    fire(u + 1, 1 - b)

                drain(u, b)
                compute(u, b)
            return carry

        lax.fori_loop(0, nunits // 2, pair_body, 0)

        pltpu.sync_copy(pos_buf, pos_out.at[pl.ds(base, bpw)])
        pltpu.sync_copy(negt_buf,
                        negt_out.at[pl.ds(wid * nunits * neg * _L,
                                          nunits * neg * _L)])

    return sc_scores


@functools.cache
def _build_tc_loss(batch, neg):
    def body(pos_ref, neg_ref, out_ref):
        p = pos_ref[...]
        s = neg_ref[...]
        sp_p = jnp.maximum(-p, 0.0) + jnp.log(1.0 + jnp.exp(-jnp.abs(p)))
        sp_n = jnp.maximum(s, 0.0) + jnp.log(1.0 + jnp.exp(-jnp.abs(s)))
        val = (jnp.sum(sp_p) * (1.0 / batch)
               + jnp.sum(sp_n) * (1.0 / (batch * neg)))
        out_ref[...] = val.reshape(1, 1)

    return pl.pallas_call(
        body,
        out_shape=jax.ShapeDtypeStruct((1, 1), jnp.float32),
    )


@jax.jit
def kernel(embeddings, output_weights, context, target, neg_samples):
    vocab, d = embeddings.shape
    batch, ctx = context.shape
    neg = neg_samples.shape[1]
    sc = _build_sc_scores(vocab, d, batch, ctx, neg)
    tc = _build_tc_loss(batch, neg)
    pos, negt = sc(embeddings, output_weights,
                   *[context[:, k] for k in range(ctx)], target,
                   *[neg_samples[:, n] for n in range(neg)])
    out = tc(pos.reshape(-1, 128), negt.reshape(-1, 128))
    return out[0, 0]
